# baseline (speedup 1.0000x reference)
import functools

import jax
import jax.numpy as jnp
from jax import lax
from jax.experimental import pallas as pl
from jax.experimental.pallas import tpu as pltpu
from jax.experimental.pallas import tpu_sc as plsc

B = 16384
D = 128
H = 2048

_NC, _NS = 2, 16
_NW = _NC * _NS
_BPW = B // _NW


def _gather_body(user_table, item_table, uids, iids, u_out, i_out,
                 idx_v, rows_v, sem):
    wid = lax.axis_index("s") * _NC + lax.axis_index("c")
    base = wid * _BPW
    pltpu.sync_copy(uids.at[pl.ds(base, _BPW)], idx_v)
    pltpu.async_copy(user_table.at[idx_v], rows_v, sem).wait()
    pltpu.sync_copy(rows_v, u_out.at[pl.ds(base, _BPW)])
    pltpu.sync_copy(iids.at[pl.ds(base, _BPW)], idx_v)
    pltpu.async_copy(item_table.at[idx_v], rows_v, sem).wait()
    pltpu.sync_copy(rows_v, i_out.at[pl.ds(base, _BPW)])


@functools.cache
def _make_gather():
    return pl.kernel(
        _gather_body,
        out_type=(jax.ShapeDtypeStruct((B, D), jnp.float32),
                  jax.ShapeDtypeStruct((B, D), jnp.float32)),
        mesh=plsc.VectorSubcoreMesh(core_axis_name="c", subcore_axis_name="s",
                                    num_cores=_NC, num_subcores=_NS),
        scratch_types=[
            pltpu.VMEM((_BPW,), jnp.int32),
            pltpu.VMEM((_BPW, D), jnp.float32),
            pltpu.SemaphoreType.DMA,
        ],
    )

_BM = 2048


def _dot(a, b):
    return jnp.dot(a.astype(jnp.bfloat16), b,
                   preferred_element_type=jnp.float32)


def _mlp_body(u_ref, i_ref, w1u_ref, w1i_ref, b1_ref, w2_ref, b2_ref,
              w3_ref, b3_ref, w4_ref, b4_ref, o_ref):
    h = _dot(u_ref[...], w1u_ref[...])
    h = h + _dot(i_ref[...], w1i_ref[...])
    h = jnp.maximum(h + b1_ref[...], 0.0)
    enc = _dot(h, w2_ref[...]) + b2_ref[...]
    h2 = jnp.maximum(_dot(enc, w3_ref[...]) + b3_ref[...], 0.0)
    o_ref[...] = _dot(h2, w4_ref[...]) + b4_ref[...]


def _mlp(u_emb, i_emb, w1u, w1i, b1, w2, b2, w3, b3, w4t, b4):
    grid = (B // _BM,)
    full = lambda shape: pl.BlockSpec(shape, lambda i: (0, 0))
    return pl.pallas_call(
        _mlp_body,
        grid=grid,
        in_specs=[
            pl.BlockSpec((_BM, D), lambda i: (i, 0)),
            pl.BlockSpec((_BM, D), lambda i: (i, 0)),
            full((D, H)),
            full((D, H)),
            full((1, H)),
            full((H, 2 * D)),
            full((1, 2 * D)),
            full((2 * D, H)),
            full((1, H)),
            full((H, 1)),
            full((1, 1)),
        ],
        out_specs=pl.BlockSpec((_BM, 1), lambda i: (i, 0)),
        out_shape=jax.ShapeDtypeStruct((B, 1), jnp.float32),
    )(u_emb, i_emb, w1u, w1i, b1, w2, b2, w3, b3, w4t, b4)


def kernel(users_ids, itens_ids, user_table, item_table,
           W1, b1, W2, b2, W3, b3, W4, b4):
    uids = users_ids.astype(jnp.int32)
    iids = itens_ids.astype(jnp.int32)
    u_emb, i_emb = _make_gather()(user_table, item_table, uids, iids)
    bf = jnp.bfloat16
    out = _mlp(u_emb, i_emb,
               W1[:D].astype(bf), W1[D:].astype(bf), b1.reshape(1, H),
               W2.astype(bf), b2.reshape(1, 2 * D),
               W3.astype(bf), b3.reshape(1, H),
               W4.astype(bf), b4.reshape(1, 1))
    return out.reshape(B)


# speedup vs baseline: 1.2700x; 1.2700x over previous
import functools

import jax
import jax.numpy as jnp
from jax import lax
from jax.experimental import pallas as pl
from jax.experimental.pallas import tpu as pltpu
from jax.experimental.pallas import tpu_sc as plsc

B = 16384
D = 128
H = 2048

_NC, _NS = 2, 16
_NW = _NC * _NS
_BPW = B // _NW


def _gather_body(user_table, item_table, uids, iids, x_out,
                 idx_v, rows_v, sem):
    wid = lax.axis_index("s") * _NC + lax.axis_index("c")
    base = wid * _BPW
    pltpu.sync_copy(uids.at[pl.ds(base, _BPW)], idx_v)
    pltpu.async_copy(user_table.at[idx_v], rows_v, sem).wait()
    pltpu.sync_copy(rows_v, x_out.at[pl.ds(base, _BPW), pl.ds(0, D)])
    pltpu.sync_copy(iids.at[pl.ds(base, _BPW)], idx_v)
    pltpu.async_copy(item_table.at[idx_v], rows_v, sem).wait()
    pltpu.sync_copy(rows_v, x_out.at[pl.ds(base, _BPW), pl.ds(D, D)])


@functools.cache
def _make_gather():
    return pl.kernel(
        _gather_body,
        out_type=jax.ShapeDtypeStruct((B, 2 * D), jnp.float32),
        mesh=plsc.VectorSubcoreMesh(core_axis_name="c", subcore_axis_name="s",
                                    num_cores=_NC, num_subcores=_NS),
        scratch_types=[
            pltpu.VMEM((_BPW,), jnp.int32),
            pltpu.VMEM((_BPW, D), jnp.float32),
            pltpu.SemaphoreType.DMA,
        ],
    )

_BM = 4096


def _dot(a, b):
    return jnp.dot(a.astype(jnp.bfloat16), b,
                   preferred_element_type=jnp.float32)


def _mlp_body(x_ref, w1_ref, b1_ref, w2_ref, b2_ref,
              w3_ref, b3_ref, w4_ref, b4_ref, o_ref):
    h = jnp.maximum(_dot(x_ref[...], w1_ref[...]) + b1_ref[...], 0.0)
    enc = _dot(h, w2_ref[...]) + b2_ref[...]
    h2 = jnp.maximum(_dot(enc, w3_ref[...]) + b3_ref[...], 0.0)
    o_ref[...] = _dot(h2, w4_ref[...]) + b4_ref[...]


def _mlp(x, w1, b1, w2, b2, w3, b3, w4, b4):
    grid = (B // _BM,)
    full = lambda shape: pl.BlockSpec(shape, lambda i: (0, 0))
    return pl.pallas_call(
        _mlp_body,
        grid=grid,
        in_specs=[
            pl.BlockSpec((_BM, 2 * D), lambda i: (i, 0)),
            full((2 * D, H)),
            full((1, H)),
            full((H, 2 * D)),
            full((1, 2 * D)),
            full((2 * D, H)),
            full((1, H)),
            full((H, 1)),
            full((1, 1)),
        ],
        out_specs=pl.BlockSpec((_BM, 1), lambda i: (i, 0)),
        out_shape=jax.ShapeDtypeStruct((B, 1), jnp.float32),
    )(x, w1, b1, w2, b2, w3, b3, w4, b4)


def kernel(users_ids, itens_ids, user_table, item_table,
           W1, b1, W2, b2, W3, b3, W4, b4):
    uids = users_ids.astype(jnp.int32)
    iids = itens_ids.astype(jnp.int32)
    x = _make_gather()(user_table, item_table, uids, iids)
    bf = jnp.bfloat16
    out = _mlp(x,
               W1.astype(bf), b1.reshape(1, H),
               W2.astype(bf), b2.reshape(1, 2 * D),
               W3.astype(bf), b3.reshape(1, H),
               W4.astype(bf), b4.reshape(1, 1))
    return out.reshape(B)
